# baseline (speedup 1.0000x reference)
import jax
import jax.numpy as jnp
from jax.experimental import pallas as pl
from jax.experimental.pallas import tpu as pltpu

_F32 = jnp.float32


def _mm_packed_kernel(a_ref, b_ref, bias_ref, o_ref):
    res = (jnp.dot(a_ref[...], b_ref[...], preferred_element_type=_F32)
           + bias_ref[...])
    for r in range(o_ref.shape[1]):
        o_ref[:, r, :] = res[:, 128 * r:128 * (r + 1)]


def _matmul_packed(a, b, bias, batch, seq, bm=512):
    m, k = a.shape
    n = b.shape[1]
    sb = seq // bm
    return pl.pallas_call(
        _mm_packed_kernel,
        grid=(m // bm,),
        in_specs=[
            pl.BlockSpec((bm, k), lambda i: (i, 0)),
            pl.BlockSpec((k, n), lambda i: (0, 0)),
            pl.BlockSpec((1, n), lambda i: (0, 0)),
        ],
        out_specs=pl.BlockSpec((None, bm, n // 128, 128),
                               lambda i: (i // sb, i % sb, 0, 0)),
        out_shape=jax.ShapeDtypeStruct((batch, seq, n // 128, 128), _F32),
        compiler_params=pltpu.CompilerParams(
            dimension_semantics=("parallel",),
        ),
    )(a, b, bias.reshape(1, n))


def _gif_kernel(cur_ref, out_ref, mem_ref):
    @pl.when(pl.program_id(0) == 0)
    def _():
        mem_ref[...] = jnp.zeros_like(mem_ref)

    cs = cur_ref.shape[1]

    def body(t, mem):
        m = 0.9 * mem + cur_ref[:, t]
        spike = 1.0 / (1.0 + jnp.exp(4.0 - 4.0 * m))
        out_ref[:, t] = spike
        return m - spike

    mem_ref[...] = jax.lax.fori_loop(0, cs, body, mem_ref[...], unroll=16)


def _gif_scan(cur_packed, cs=512):
    b, s, r, l = cur_packed.shape
    return pl.pallas_call(
        _gif_kernel,
        grid=(s // cs,),
        in_specs=[
            pl.BlockSpec((b, cs, r, l), lambda j: (0, j, 0, 0)),
        ],
        out_specs=pl.BlockSpec((b, cs, r, l), lambda j: (0, j, 0, 0)),
        out_shape=jax.ShapeDtypeStruct((b, s, r, l), _F32),
        scratch_shapes=[pltpu.VMEM((b, r, l), _F32)],
        compiler_params=pltpu.CompilerParams(
            dimension_semantics=("arbitrary",),
        ),
    )(cur_packed)


def _moe_kernel(sp_ref, w_s2c_ref, b_s2c_ref, wr1_ref, br1_ref, wr2_ref,
                br2_ref, w1cat_ref, be1cat_ref, w2cat_ref, be2_ref,
                w_c2s_ref, b_c2s_ref, w_dec_ref, b_dec_ref, o_ref):
    rt = sp_ref.shape[1]
    cont = b_s2c_ref[...]
    for r in range(rt):
        cont = cont + jnp.dot(sp_ref[:, r, :],
                              w_s2c_ref[128 * r:128 * (r + 1), :],
                              preferred_element_type=_F32)
    h = jnp.tanh(jnp.dot(cont, wr1_ref[...], preferred_element_type=_F32)
                 + br1_ref[...])
    logits = (jnp.dot(h, wr2_ref[...], preferred_element_type=_F32)
              + br2_ref[...])
    bt, ne = logits.shape
    exp_h = w1cat_ref.shape[1] // ne
    eidx = jax.lax.broadcasted_iota(jnp.int32, (bt, ne), 1)
    v1 = jnp.max(logits, axis=-1, keepdims=True)
    i1 = jnp.min(jnp.where(logits == v1, eidx, ne), axis=-1, keepdims=True)
    oh1 = eidx == i1
    l2 = jnp.where(oh1, jnp.float32(-1e30), logits)
    v2 = jnp.max(l2, axis=-1, keepdims=True)
    i2 = jnp.min(jnp.where(l2 == v2, eidx, ne), axis=-1, keepdims=True)
    oh2 = eidx == i2
    w1 = 1.0 / (1.0 + jnp.exp(v2 - v1))
    gate = jnp.where(oh1, w1, 0.0) + jnp.where(oh2, 1.0 - w1, 0.0)
    acc = jnp.dot(gate, be2_ref[...], preferred_element_type=_F32)
    for e in range(ne):
        h1 = jnp.maximum(
            jnp.dot(cont, w1cat_ref[:, exp_h * e:exp_h * (e + 1)],
                    preferred_element_type=_F32)
            + be1cat_ref[:, exp_h * e:exp_h * (e + 1)], 0.0)
        acc = acc + gate[:, e:e + 1] * jnp.dot(
            h1, w2cat_ref[exp_h * e:exp_h * (e + 1), :],
            preferred_element_type=_F32)
    moe_sig = jax.nn.sigmoid(
        jnp.dot(acc, w_c2s_ref[...], preferred_element_type=_F32)
        + b_c2s_ref[...])
    cur = (jnp.dot(moe_sig, w_dec_ref[...], preferred_element_type=_F32)
           + b_dec_ref[...])
    for r in range(o_ref.shape[1]):
        o_ref[:, r, :] = cur[:, 128 * r:128 * (r + 1)]


def _moe_dec(sp_packed, W_s2c, b_s2c, Wr1, br1, Wr2, br2, We1, be1, We2, be2,
             W_c2s, b_c2s, W_dec, b_dec, bt=512):
    batch, seq, rt, l = sp_packed.shape
    hid = rt * l
    moe_h = W_s2c.shape[1]
    ne = We1.shape[0]
    exp_h = We1.shape[2]
    d_out = W_dec.shape[1]
    sb = seq // bt
    w1cat = We1.transpose(1, 0, 2).reshape(moe_h, ne * exp_h)
    w2cat = We2.reshape(ne * exp_h, moe_h)
    full = lambda shape: pl.BlockSpec(shape, lambda i: tuple(0 for _ in shape))
    return pl.pallas_call(
        _moe_kernel,
        grid=(batch * sb,),
        in_specs=[
            pl.BlockSpec((None, bt, rt, l), lambda i: (i // sb, i % sb, 0, 0)),
            full((hid, moe_h)),
            full((1, moe_h)),
            full((moe_h, Wr1.shape[1])),
            full((1, br1.shape[0])),
            full((Wr2.shape[0], ne)),
            full((1, ne)),
            full((moe_h, ne * exp_h)),
            full((1, ne * exp_h)),
            full((ne * exp_h, moe_h)),
            full((ne, moe_h)),
            full((moe_h, hid)),
            full((1, hid)),
            full((hid, d_out)),
            full((1, d_out)),
        ],
        out_specs=pl.BlockSpec((None, bt, d_out // 128, 128),
                               lambda i: (i // sb, i % sb, 0, 0)),
        out_shape=jax.ShapeDtypeStruct((batch, seq, d_out // 128, 128), _F32),
        compiler_params=pltpu.CompilerParams(
            dimension_semantics=("arbitrary",),
        ),
    )(sp_packed, W_s2c, b_s2c.reshape(1, -1), Wr1, br1.reshape(1, -1), Wr2,
      br2.reshape(1, -1), w1cat, be1.reshape(1, -1), w2cat, be2,
      W_c2s, b_c2s.reshape(1, -1), W_dec, b_dec.reshape(1, -1))


def _ln_kernel(x_ref, g_ref, b_ref, o_ref):
    rt = x_ref.shape[1]
    d = rt * x_ref.shape[2]
    s = jnp.zeros((x_ref.shape[0], 1), _F32)
    for r in range(rt):
        s = s + jnp.sum(x_ref[:, r, :], axis=-1, keepdims=True)
    mu = s / d
    ss = jnp.zeros((x_ref.shape[0], 1), _F32)
    for r in range(rt):
        xc = x_ref[:, r, :] - mu
        ss = ss + jnp.sum(xc * xc, axis=-1, keepdims=True)
    inv = jax.lax.rsqrt(ss / d + 1e-5)
    for r in range(rt):
        o_ref[:, 128 * r:128 * (r + 1)] = (
            (x_ref[:, r, :] - mu) * inv * g_ref[:, 128 * r:128 * (r + 1)]
            + b_ref[:, 128 * r:128 * (r + 1)])


def _layernorm(x_packed, g, b, bt=512):
    batch, seq, rt, l = x_packed.shape
    d = rt * l
    sb = seq // bt
    return pl.pallas_call(
        _ln_kernel,
        grid=(batch * sb,),
        in_specs=[
            pl.BlockSpec((None, bt, rt, l), lambda i: (i // sb, i % sb, 0, 0)),
            pl.BlockSpec((1, d), lambda i: (0, 0)),
            pl.BlockSpec((1, d), lambda i: (0, 0)),
        ],
        out_specs=pl.BlockSpec((bt, d), lambda i: (i, 0)),
        out_shape=jax.ShapeDtypeStruct((batch * seq, d), _F32),
        compiler_params=pltpu.CompilerParams(
            dimension_semantics=("parallel",),
        ),
    )(x_packed, g.reshape(1, d), b.reshape(1, d))


def kernel(inputs_embeds, W_enc, b_enc, W_s2c, b_s2c, Wr1, br1, Wr2, br2,
           We1, be1, We2, be2, W_c2s, b_c2s, W_dec, b_dec, ln_g, ln_b):
    b, s, d = inputs_embeds.shape
    n = b * s
    flat = inputs_embeds.reshape(n, d)
    cur_enc = _matmul_packed(flat, W_enc, b_enc, b, s)
    spikes = _gif_scan(cur_enc)
    cur_dec = _moe_dec(spikes, W_s2c, b_s2c, Wr1, br1, Wr2, br2, We1, be1,
                       We2, be2, W_c2s, b_c2s, W_dec, b_dec)
    decoded = _gif_scan(cur_dec)
    out = _layernorm(decoded, ln_g, ln_b)
    return out.reshape(b, s, d)


# speedup vs baseline: 1.0855x; 1.0855x over previous
import jax
import jax.numpy as jnp
from jax.experimental import pallas as pl
from jax.experimental.pallas import tpu as pltpu

_F32 = jnp.float32


def _mm_packed_kernel(a_ref, b_ref, bias_ref, o_ref):
    res = 2.0 * (jnp.dot(a_ref[...], b_ref[...], preferred_element_type=_F32)
                 + bias_ref[...])
    for r in range(o_ref.shape[1]):
        o_ref[:, r, :] = res[:, 128 * r:128 * (r + 1)]


def _matmul_packed(a, b, bias, batch, seq, bm=512):
    m, k = a.shape
    n = b.shape[1]
    sb = seq // bm
    return pl.pallas_call(
        _mm_packed_kernel,
        grid=(m // bm,),
        in_specs=[
            pl.BlockSpec((bm, k), lambda i: (i, 0)),
            pl.BlockSpec((k, n), lambda i: (0, 0)),
            pl.BlockSpec((1, n), lambda i: (0, 0)),
        ],
        out_specs=pl.BlockSpec((None, bm, n // 128, 128),
                               lambda i: (i // sb, i % sb, 0, 0)),
        out_shape=jax.ShapeDtypeStruct((batch, seq, n // 128, 128), _F32),
        compiler_params=pltpu.CompilerParams(
            dimension_semantics=("parallel",),
        ),
    )(a, b, bias.reshape(1, n))


def _gif_kernel(cur_ref, out_ref, mem_ref):
    @pl.when(pl.program_id(0) == 0)
    def _():
        mem_ref[...] = jnp.zeros_like(mem_ref)

    cs = cur_ref.shape[1]

    def body(t, y):
        y = 0.9 * y + cur_ref[:, t]
        spike = 0.5 + 0.5 * jnp.tanh(y - 2.0)
        out_ref[:, t] = spike
        return y - 2.0 * spike

    mem_ref[...] = jax.lax.fori_loop(0, cs, body, mem_ref[...], unroll=8)


def _gif_scan(cur_packed, cs=256):
    b, s, r, l = cur_packed.shape
    return pl.pallas_call(
        _gif_kernel,
        grid=(s // cs,),
        in_specs=[
            pl.BlockSpec((b, cs, r, l), lambda j: (0, j, 0, 0)),
        ],
        out_specs=pl.BlockSpec((b, cs, r, l), lambda j: (0, j, 0, 0)),
        out_shape=jax.ShapeDtypeStruct((b, s, r, l), _F32),
        scratch_shapes=[pltpu.VMEM((b, r, l), _F32)],
        compiler_params=pltpu.CompilerParams(
            dimension_semantics=("arbitrary",),
        ),
    )(cur_packed)


def _moe_kernel(sp_ref, w_s2c_ref, b_s2c_ref, wr1_ref, br1_ref, wr2_ref,
                br2_ref, w1cat_ref, be1cat_ref, w2cat_ref, be2_ref,
                w_c2s_ref, b_c2s_ref, w_dec_ref, b_dec_ref, o_ref):
    rt = sp_ref.shape[1]
    cont = b_s2c_ref[...]
    for r in range(rt):
        cont = cont + jnp.dot(sp_ref[:, r, :],
                              w_s2c_ref[128 * r:128 * (r + 1), :],
                              preferred_element_type=_F32)
    h = jnp.tanh(jnp.dot(cont, wr1_ref[...], preferred_element_type=_F32)
                 + br1_ref[...])
    logits = (jnp.dot(h, wr2_ref[...], preferred_element_type=_F32)
              + br2_ref[...])
    bt, ne = logits.shape
    exp_h = w1cat_ref.shape[1] // ne
    eidx = jax.lax.broadcasted_iota(jnp.int32, (bt, ne), 1)
    v1 = jnp.max(logits, axis=-1, keepdims=True)
    i1 = jnp.min(jnp.where(logits == v1, eidx, ne), axis=-1, keepdims=True)
    oh1 = eidx == i1
    l2 = jnp.where(oh1, jnp.float32(-1e30), logits)
    v2 = jnp.max(l2, axis=-1, keepdims=True)
    i2 = jnp.min(jnp.where(l2 == v2, eidx, ne), axis=-1, keepdims=True)
    oh2 = eidx == i2
    w1 = 1.0 / (1.0 + jnp.exp(v2 - v1))
    gate = jnp.where(oh1, w1, 0.0) + jnp.where(oh2, 1.0 - w1, 0.0)
    acc = jnp.dot(gate, be2_ref[...], preferred_element_type=_F32)
    for e in range(ne):
        h1 = jnp.maximum(
            jnp.dot(cont, w1cat_ref[:, exp_h * e:exp_h * (e + 1)],
                    preferred_element_type=_F32)
            + be1cat_ref[:, exp_h * e:exp_h * (e + 1)], 0.0)
        acc = acc + gate[:, e:e + 1] * jnp.dot(
            h1, w2cat_ref[exp_h * e:exp_h * (e + 1), :],
            preferred_element_type=_F32)
    moe_sig = jax.nn.sigmoid(
        jnp.dot(acc, w_c2s_ref[...], preferred_element_type=_F32)
        + b_c2s_ref[...])
    cur = 2.0 * (jnp.dot(moe_sig, w_dec_ref[...], preferred_element_type=_F32)
                 + b_dec_ref[...])
    for r in range(o_ref.shape[1]):
        o_ref[:, r, :] = cur[:, 128 * r:128 * (r + 1)]


def _moe_dec(sp_packed, W_s2c, b_s2c, Wr1, br1, Wr2, br2, We1, be1, We2, be2,
             W_c2s, b_c2s, W_dec, b_dec, bt=512):
    batch, seq, rt, l = sp_packed.shape
    hid = rt * l
    moe_h = W_s2c.shape[1]
    ne = We1.shape[0]
    exp_h = We1.shape[2]
    d_out = W_dec.shape[1]
    sb = seq // bt
    w1cat = We1.transpose(1, 0, 2).reshape(moe_h, ne * exp_h)
    w2cat = We2.reshape(ne * exp_h, moe_h)
    full = lambda shape: pl.BlockSpec(shape, lambda i: tuple(0 for _ in shape))
    return pl.pallas_call(
        _moe_kernel,
        grid=(batch * sb,),
        in_specs=[
            pl.BlockSpec((None, bt, rt, l), lambda i: (i // sb, i % sb, 0, 0)),
            full((hid, moe_h)),
            full((1, moe_h)),
            full((moe_h, Wr1.shape[1])),
            full((1, br1.shape[0])),
            full((Wr2.shape[0], ne)),
            full((1, ne)),
            full((moe_h, ne * exp_h)),
            full((1, ne * exp_h)),
            full((ne * exp_h, moe_h)),
            full((ne, moe_h)),
            full((moe_h, hid)),
            full((1, hid)),
            full((hid, d_out)),
            full((1, d_out)),
        ],
        out_specs=pl.BlockSpec((None, bt, d_out // 128, 128),
                               lambda i: (i // sb, i % sb, 0, 0)),
        out_shape=jax.ShapeDtypeStruct((batch, seq, d_out // 128, 128), _F32),
        compiler_params=pltpu.CompilerParams(
            dimension_semantics=("arbitrary",),
        ),
    )(sp_packed, W_s2c, b_s2c.reshape(1, -1), Wr1, br1.reshape(1, -1), Wr2,
      br2.reshape(1, -1), w1cat, be1.reshape(1, -1), w2cat, be2,
      W_c2s, b_c2s.reshape(1, -1), W_dec, b_dec.reshape(1, -1))


def _ln_kernel(x_ref, g_ref, b_ref, o_ref):
    rt = x_ref.shape[1]
    d = rt * x_ref.shape[2]
    s = jnp.zeros((x_ref.shape[0], 1), _F32)
    for r in range(rt):
        s = s + jnp.sum(x_ref[:, r, :], axis=-1, keepdims=True)
    mu = s / d
    ss = jnp.zeros((x_ref.shape[0], 1), _F32)
    for r in range(rt):
        xc = x_ref[:, r, :] - mu
        ss = ss + jnp.sum(xc * xc, axis=-1, keepdims=True)
    inv = jax.lax.rsqrt(ss / d + 1e-5)
    for r in range(rt):
        o_ref[:, 128 * r:128 * (r + 1)] = (
            (x_ref[:, r, :] - mu) * inv * g_ref[:, 128 * r:128 * (r + 1)]
            + b_ref[:, 128 * r:128 * (r + 1)])


def _layernorm(x_packed, g, b, bt=512):
    batch, seq, rt, l = x_packed.shape
    d = rt * l
    sb = seq // bt
    return pl.pallas_call(
        _ln_kernel,
        grid=(batch * sb,),
        in_specs=[
            pl.BlockSpec((None, bt, rt, l), lambda i: (i // sb, i % sb, 0, 0)),
            pl.BlockSpec((1, d), lambda i: (0, 0)),
            pl.BlockSpec((1, d), lambda i: (0, 0)),
        ],
        out_specs=pl.BlockSpec((bt, d), lambda i: (i, 0)),
        out_shape=jax.ShapeDtypeStruct((batch * seq, d), _F32),
        compiler_params=pltpu.CompilerParams(
            dimension_semantics=("parallel",),
        ),
    )(x_packed, g.reshape(1, d), b.reshape(1, d))


def kernel(inputs_embeds, W_enc, b_enc, W_s2c, b_s2c, Wr1, br1, Wr2, br2,
           We1, be1, We2, be2, W_c2s, b_c2s, W_dec, b_dec, ln_g, ln_b):
    b, s, d = inputs_embeds.shape
    n = b * s
    flat = inputs_embeds.reshape(n, d)
    cur_enc = _matmul_packed(flat, W_enc, b_enc, b, s)
    spikes = _gif_scan(cur_enc)
    cur_dec = _moe_dec(spikes, W_s2c, b_s2c, Wr1, br1, Wr2, br2, We1, be1,
                       We2, be2, W_c2s, b_c2s, W_dec, b_dec)
    decoded = _gif_scan(cur_dec)
    out = _layernorm(decoded, ln_g, ln_b)
    return out.reshape(b, s, d)


# speedup vs baseline: 1.0924x; 1.0064x over previous
import jax
import jax.numpy as jnp
from jax.experimental import pallas as pl
from jax.experimental.pallas import tpu as pltpu

_F32 = jnp.float32


def _mm_packed_kernel(a_ref, b_ref, bias_ref, o_ref):
    res = 2.0 * (jnp.dot(a_ref[...], b_ref[...], preferred_element_type=_F32)
                 + bias_ref[...])
    for r in range(o_ref.shape[1]):
        o_ref[:, r, :] = res[:, 128 * r:128 * (r + 1)]


def _matmul_packed(a, b, bias, batch, seq, bm=512):
    m, k = a.shape
    n = b.shape[1]
    sb = seq // bm
    return pl.pallas_call(
        _mm_packed_kernel,
        grid=(m // bm,),
        in_specs=[
            pl.BlockSpec((bm, k), lambda i: (i, 0)),
            pl.BlockSpec((k, n), lambda i: (0, 0)),
            pl.BlockSpec((1, n), lambda i: (0, 0)),
        ],
        out_specs=pl.BlockSpec((None, bm, n // 128, 128),
                               lambda i: (i // sb, i % sb, 0, 0)),
        out_shape=jax.ShapeDtypeStruct((batch, seq, n // 128, 128), _F32),
        compiler_params=pltpu.CompilerParams(
            dimension_semantics=("parallel",),
        ),
    )(a, b, bias.reshape(1, n))


def _gif_kernel(cur_ref, out_ref, mem_ref):
    @pl.when(pl.program_id(0) == 0)
    def _():
        mem_ref[...] = jnp.zeros_like(mem_ref)

    cs = cur_ref.shape[1]

    def body(t, y):
        y = 0.9 * y + cur_ref[:, t]
        spike = 0.5 + 0.5 * jnp.tanh(y - 2.0)
        out_ref[:, t] = spike
        return y - 2.0 * spike

    mem_ref[...] = jax.lax.fori_loop(0, cs, body, mem_ref[...], unroll=8)


def _gif_scan(cur_packed, cs=256):
    b, s, r, l = cur_packed.shape
    return pl.pallas_call(
        _gif_kernel,
        grid=(s // cs,),
        in_specs=[
            pl.BlockSpec((b, cs, r, l), lambda j: (0, j, 0, 0)),
        ],
        out_specs=pl.BlockSpec((b, cs, r, l), lambda j: (0, j, 0, 0)),
        out_shape=jax.ShapeDtypeStruct((b, s, r, l), _F32),
        scratch_shapes=[pltpu.VMEM((b, r, l), _F32)],
        compiler_params=pltpu.CompilerParams(
            dimension_semantics=("arbitrary",),
        ),
    )(cur_packed)


def _moe_kernel(sp_ref, w_s2c_ref, b_s2c_ref, wr1_ref, br1_ref, wr2_ref,
                br2_ref, w1cat_ref, be1cat_ref, w2cat_ref, be2_ref,
                w_c2s_ref, b_c2s_ref, w_dec_ref, b_dec_ref, o_ref):
    rt = sp_ref.shape[1]
    cont = b_s2c_ref[...]
    for r in range(rt):
        cont = cont + jnp.dot(sp_ref[:, r, :],
                              w_s2c_ref[128 * r:128 * (r + 1), :],
                              preferred_element_type=_F32)
    h = jnp.tanh(jnp.dot(cont, wr1_ref[...], preferred_element_type=_F32)
                 + br1_ref[...])
    logits = (jnp.dot(h, wr2_ref[...], preferred_element_type=_F32)
              + br2_ref[...])
    bt, ne = logits.shape
    exp_h = w1cat_ref.shape[1] // ne
    eidx = jax.lax.broadcasted_iota(jnp.int32, (bt, ne), 1)
    v1 = jnp.max(logits, axis=-1, keepdims=True)
    i1 = jnp.min(jnp.where(logits == v1, eidx, ne), axis=-1, keepdims=True)
    oh1 = eidx == i1
    l2 = jnp.where(oh1, jnp.float32(-1e30), logits)
    v2 = jnp.max(l2, axis=-1, keepdims=True)
    i2 = jnp.min(jnp.where(l2 == v2, eidx, ne), axis=-1, keepdims=True)
    oh2 = eidx == i2
    w1 = 1.0 / (1.0 + jnp.exp(v2 - v1))
    gate = jnp.where(oh1, w1, 0.0) + jnp.where(oh2, 1.0 - w1, 0.0)
    acc = jnp.dot(gate, be2_ref[...], preferred_element_type=_F32)
    for e in range(ne):
        h1 = jnp.maximum(
            jnp.dot(cont, w1cat_ref[:, exp_h * e:exp_h * (e + 1)],
                    preferred_element_type=_F32)
            + be1cat_ref[:, exp_h * e:exp_h * (e + 1)], 0.0)
        acc = acc + gate[:, e:e + 1] * jnp.dot(
            h1, w2cat_ref[exp_h * e:exp_h * (e + 1), :],
            preferred_element_type=_F32)
    moe_sig = jax.nn.sigmoid(
        jnp.dot(acc, w_c2s_ref[...], preferred_element_type=_F32)
        + b_c2s_ref[...])
    cur = 2.0 * (jnp.dot(moe_sig, w_dec_ref[...], preferred_element_type=_F32)
                 + b_dec_ref[...])
    for r in range(o_ref.shape[1]):
        o_ref[:, r, :] = cur[:, 128 * r:128 * (r + 1)]


def _moe_dec(sp_packed, W_s2c, b_s2c, Wr1, br1, Wr2, br2, We1, be1, We2, be2,
             W_c2s, b_c2s, W_dec, b_dec, bt=1024):
    batch, seq, rt, l = sp_packed.shape
    hid = rt * l
    moe_h = W_s2c.shape[1]
    ne = We1.shape[0]
    exp_h = We1.shape[2]
    d_out = W_dec.shape[1]
    sb = seq // bt
    w1cat = We1.transpose(1, 0, 2).reshape(moe_h, ne * exp_h)
    w2cat = We2.reshape(ne * exp_h, moe_h)
    full = lambda shape: pl.BlockSpec(shape, lambda i: tuple(0 for _ in shape))
    return pl.pallas_call(
        _moe_kernel,
        grid=(batch * sb,),
        in_specs=[
            pl.BlockSpec((None, bt, rt, l), lambda i: (i // sb, i % sb, 0, 0)),
            full((hid, moe_h)),
            full((1, moe_h)),
            full((moe_h, Wr1.shape[1])),
            full((1, br1.shape[0])),
            full((Wr2.shape[0], ne)),
            full((1, ne)),
            full((moe_h, ne * exp_h)),
            full((1, ne * exp_h)),
            full((ne * exp_h, moe_h)),
            full((ne, moe_h)),
            full((moe_h, hid)),
            full((1, hid)),
            full((hid, d_out)),
            full((1, d_out)),
        ],
        out_specs=pl.BlockSpec((None, bt, d_out // 128, 128),
                               lambda i: (i // sb, i % sb, 0, 0)),
        out_shape=jax.ShapeDtypeStruct((batch, seq, d_out // 128, 128), _F32),
        compiler_params=pltpu.CompilerParams(
            dimension_semantics=("arbitrary",),
        ),
    )(sp_packed, W_s2c, b_s2c.reshape(1, -1), Wr1, br1.reshape(1, -1), Wr2,
      br2.reshape(1, -1), w1cat, be1.reshape(1, -1), w2cat, be2,
      W_c2s, b_c2s.reshape(1, -1), W_dec, b_dec.reshape(1, -1))


def _ln_kernel(x_ref, g_ref, b_ref, o_ref):
    rt = x_ref.shape[1]
    d = rt * x_ref.shape[2]
    s = jnp.zeros((x_ref.shape[0], 1), _F32)
    for r in range(rt):
        s = s + jnp.sum(x_ref[:, r, :], axis=-1, keepdims=True)
    mu = s / d
    ss = jnp.zeros((x_ref.shape[0], 1), _F32)
    for r in range(rt):
        xc = x_ref[:, r, :] - mu
        ss = ss + jnp.sum(xc * xc, axis=-1, keepdims=True)
    inv = jax.lax.rsqrt(ss / d + 1e-5)
    for r in range(rt):
        o_ref[:, 128 * r:128 * (r + 1)] = (
            (x_ref[:, r, :] - mu) * inv * g_ref[:, 128 * r:128 * (r + 1)]
            + b_ref[:, 128 * r:128 * (r + 1)])


def _layernorm(x_packed, g, b, bt=512):
    batch, seq, rt, l = x_packed.shape
    d = rt * l
    sb = seq // bt
    return pl.pallas_call(
        _ln_kernel,
        grid=(batch * sb,),
        in_specs=[
            pl.BlockSpec((None, bt, rt, l), lambda i: (i // sb, i % sb, 0, 0)),
            pl.BlockSpec((1, d), lambda i: (0, 0)),
            pl.BlockSpec((1, d), lambda i: (0, 0)),
        ],
        out_specs=pl.BlockSpec((bt, d), lambda i: (i, 0)),
        out_shape=jax.ShapeDtypeStruct((batch * seq, d), _F32),
        compiler_params=pltpu.CompilerParams(
            dimension_semantics=("parallel",),
        ),
    )(x_packed, g.reshape(1, d), b.reshape(1, d))


def kernel(inputs_embeds, W_enc, b_enc, W_s2c, b_s2c, Wr1, br1, Wr2, br2,
           We1, be1, We2, be2, W_c2s, b_c2s, W_dec, b_dec, ln_g, ln_b):
    b, s, d = inputs_embeds.shape
    n = b * s
    flat = inputs_embeds.reshape(n, d)
    cur_enc = _matmul_packed(flat, W_enc, b_enc, b, s)
    spikes = _gif_scan(cur_enc)
    cur_dec = _moe_dec(spikes, W_s2c, b_s2c, Wr1, br1, Wr2, br2, We1, be1,
                       We2, be2, W_c2s, b_c2s, W_dec, b_dec)
    decoded = _gif_scan(cur_dec)
    out = _layernorm(decoded, ln_g, ln_b)
    return out.reshape(b, s, d)
